# baseline (speedup 1.0000x reference)
import functools

import jax
import jax.numpy as jnp
from jax import lax
from jax.experimental import pallas as pl
from jax.experimental.pallas import tpu as pltpu

_NBUF = 3
_NSPLIT = 8


def _gate_kernel(x_ref, w_ref,
                 idx_ref,
                 counts_ref, laux_ref, combine_hbm, dispatch_hbm,
                 bufs, dbufs, sems, dsems, base_ref, gsum_ref):
    i = pl.program_id(0)
    n = pl.num_programs(0)
    T = x_ref.shape[0]
    E = w_ref.shape[1]
    C = bufs.shape[3]

    @pl.when(i == 0)
    def _init():
        base_ref[...] = jnp.zeros_like(base_ref)
        gsum_ref[...] = jnp.zeros_like(gsum_ref)

    x = x_ref[...]
    w = w_ref[...]
    logits = jnp.dot(x, w, preferred_element_type=jnp.float32)

    lmax = jnp.max(logits, axis=1, keepdims=True)
    ex = jnp.exp(logits - lmax)
    gates = ex / jnp.sum(ex, axis=1, keepdims=True)

    gmax = jnp.max(gates, axis=1, keepdims=True)
    eiota = lax.broadcasted_iota(jnp.int32, (T, E), 1)
    idx = jnp.min(jnp.where(gates == gmax, eiota, E), axis=1, keepdims=True)

    mask = (eiota == idx).astype(jnp.int32)
    mask_f = mask.astype(jnp.float32)
    row = lax.broadcasted_iota(jnp.int32, (T, T), 0)
    col = lax.broadcasted_iota(jnp.int32, (T, T), 1)
    tri = (col <= row).astype(jnp.float32)
    csum = jnp.dot(tri, mask_f, preferred_element_type=jnp.float32)
    base = base_ref[...].astype(jnp.float32)
    loc = jnp.sum((csum - 1.0 + base) * mask_f,
                  axis=1, keepdims=True).astype(jnp.int32)

    new_base = base_ref[...] + jnp.sum(mask, axis=0, keepdims=True)
    base_ref[...] = new_base
    gsum_ref[...] = gsum_ref[...] + jnp.sum(gates, axis=0, keepdims=True)

    idx_ref[...] = idx
    counts_ref[...] = new_base
    S = n * T
    laux_ref[...] = jnp.sum(
        (gsum_ref[...] / S) * (new_base.astype(jnp.float32) / S),
        keepdims=True,
    ) * E

    gate_val = jnp.where(eiota == idx, gmax, 0.0)
    ciota = lax.broadcasted_iota(jnp.int32, (T, C), 1)
    slot = (ciota == loc).astype(jnp.float32)
    sl = lax.rem(i, _NBUF)
    P = T // _NSPLIT

    @pl.when(i >= _NBUF)
    def _drain():
        for p in range(_NSPLIT):
            pltpu.make_async_copy(
                bufs.at[sl, pl.ds(p * P, P)],
                combine_hbm.at[pl.ds((i - _NBUF) * T + p * P, P)],
                sems.at[sl, p],
            ).wait()
        for p in range(_NSPLIT):
            pltpu.make_async_copy(
                dbufs.at[sl, pl.ds(p * P, P)],
                dispatch_hbm.at[pl.ds((i - _NBUF) * T + p * P, P)],
                dsems.at[sl, p],
            ).wait()

    for p in range(_NSPLIT):
        rows = slice(p * P, (p + 1) * P)
        chunk = gate_val[rows, :, None] * slot[rows, None, :]
        bufs[sl, pl.ds(p * P, P)] = chunk
        dbufs[sl, pl.ds(p * P, P)] = (chunk != 0.0).astype(jnp.int8)
        pltpu.make_async_copy(
            bufs.at[sl, pl.ds(p * P, P)],
            combine_hbm.at[pl.ds(i * T + p * P, P)],
            sems.at[sl, p],
        ).start()
        pltpu.make_async_copy(
            dbufs.at[sl, pl.ds(p * P, P)],
            dispatch_hbm.at[pl.ds(i * T + p * P, P)],
            dsems.at[sl, p],
        ).start()

    @pl.when(i == n - 1)
    def _final():
        for k in range(_NBUF):
            step = n - _NBUF + k

            @pl.when(step >= 0)
            def _():
                s2 = lax.rem(jnp.int32(step), _NBUF)
                for p in range(_NSPLIT):
                    pltpu.make_async_copy(
                        bufs.at[s2, pl.ds(p * P, P)],
                        combine_hbm.at[pl.ds(step * T + p * P, P)],
                        sems.at[s2, p],
                    ).wait()
                for p in range(_NSPLIT):
                    pltpu.make_async_copy(
                        dbufs.at[s2, pl.ds(p * P, P)],
                        dispatch_hbm.at[pl.ds(step * T + p * P, P)],
                        dsems.at[s2, p],
                    ).wait()


@functools.partial(jax.jit, static_argnames=("block_t",))
def _top1_gate(x, W, block_t=512):
    S, D = x.shape
    E = W.shape[1]
    import numpy as np
    C = max(int(np.ceil(S / E * 1.0)), 4)
    n = S // block_t

    out_shapes = (
        jax.ShapeDtypeStruct((S, 1), jnp.int32),
        jax.ShapeDtypeStruct((1, E), jnp.int32),
        jax.ShapeDtypeStruct((1, 1), jnp.float32),
        jax.ShapeDtypeStruct((S, E, C), jnp.float32),
        jax.ShapeDtypeStruct((S, E, C), jnp.int8),
    )
    return pl.pallas_call(
        _gate_kernel,
        grid=(n,),
        in_specs=[
            pl.BlockSpec((block_t, D), lambda i: (i, 0)),
            pl.BlockSpec((D, E), lambda i: (0, 0)),
        ],
        out_specs=(
            pl.BlockSpec((block_t, 1), lambda i: (i, 0)),
            pl.BlockSpec((1, E), lambda i: (0, 0)),
            pl.BlockSpec((1, 1), lambda i: (0, 0)),
            pl.BlockSpec(memory_space=pl.ANY),
            pl.BlockSpec(memory_space=pl.ANY),
        ),
        out_shape=out_shapes,
        scratch_shapes=[
            pltpu.VMEM((_NBUF, block_t, E, C), jnp.float32),
            pltpu.VMEM((_NBUF, block_t, E, C), jnp.int8),
            pltpu.SemaphoreType.DMA((_NBUF, _NSPLIT)),
            pltpu.SemaphoreType.DMA((_NBUF, _NSPLIT)),
            pltpu.VMEM((1, E), jnp.int32),
            pltpu.VMEM((1, E), jnp.float32),
        ],
    )(x, W)


def kernel(input, W):
    import numpy as np
    S, D = input.shape
    E = W.shape[1]
    C = max(int(np.ceil(S / E * 1.0)), 4)

    idx2, counts, laux, combine, dispatch_i8 = _top1_gate(input, W)

    mask1 = (idx2 == jnp.arange(E, dtype=jnp.int32)[None, :]).astype(jnp.int32)

    return (laux[0, 0], combine, dispatch_i8.astype(jnp.bool_), mask1,
            counts[0], idx2[:, 0])


# speedup vs baseline: 1.0293x; 1.0293x over previous
import functools

import jax
import jax.numpy as jnp
from jax import lax
from jax.experimental import pallas as pl
from jax.experimental.pallas import tpu as pltpu

_NBUF = 3
_NSPLIT = 16


def _gate_kernel(x_ref, w_ref,
                 idx_ref,
                 counts_ref, laux_ref, combine_hbm, dispatch_hbm,
                 bufs, dbufs, sems, dsems, base_ref, gsum_ref):
    i = pl.program_id(0)
    n = pl.num_programs(0)
    T = x_ref.shape[0]
    E = w_ref.shape[0]
    C = bufs.shape[3]

    @pl.when(i == 0)
    def _init():
        base_ref[...] = jnp.zeros_like(base_ref)
        gsum_ref[...] = jnp.zeros_like(gsum_ref)

    x = x_ref[...]
    wt = w_ref[...]
    logits = lax.dot_general(x, wt, (((1,), (1,)), ((), ())),
                             preferred_element_type=jnp.float32)

    lmax = jnp.max(logits, axis=1, keepdims=True)
    ex = jnp.exp(logits - lmax)
    gates = ex / jnp.sum(ex, axis=1, keepdims=True)

    gmax = jnp.max(gates, axis=1, keepdims=True)
    eiota = lax.broadcasted_iota(jnp.int32, (T, E), 1)
    idx = jnp.min(jnp.where(gates == gmax, eiota, E), axis=1, keepdims=True)

    mask = (eiota == idx).astype(jnp.int32)
    mask_f = mask.astype(jnp.float32)
    row = lax.broadcasted_iota(jnp.int32, (T, T), 0)
    col = lax.broadcasted_iota(jnp.int32, (T, T), 1)
    tri = (col <= row).astype(jnp.float32)
    csum = jnp.dot(tri, mask_f, preferred_element_type=jnp.float32)
    base = base_ref[...].astype(jnp.float32)
    loc = jnp.sum((csum - 1.0 + base) * mask_f,
                  axis=1, keepdims=True).astype(jnp.int32)

    new_base = base_ref[...] + jnp.sum(mask, axis=0, keepdims=True)
    base_ref[...] = new_base
    gsum_ref[...] = gsum_ref[...] + jnp.sum(gates, axis=0, keepdims=True)

    idx_ref[...] = idx
    counts_ref[...] = new_base
    S = n * T
    laux_ref[...] = jnp.sum(
        (gsum_ref[...] / S) * (new_base.astype(jnp.float32) / S),
        keepdims=True,
    ) * E

    gate_val = jnp.where(eiota == idx, gmax, 0.0)
    ciota = lax.broadcasted_iota(jnp.int32, (T, C), 1)
    slot = (ciota == loc).astype(jnp.float32)
    sl = lax.rem(i, _NBUF)
    P = T // _NSPLIT

    @pl.when(i >= _NBUF)
    def _drain():
        for p in range(_NSPLIT):
            pltpu.make_async_copy(
                bufs.at[sl, pl.ds(p * P, P)],
                combine_hbm.at[pl.ds((i - _NBUF) * T + p * P, P)],
                sems.at[sl, p],
            ).wait()
        for p in range(_NSPLIT):
            pltpu.make_async_copy(
                dbufs.at[sl, pl.ds(p * P, P)],
                dispatch_hbm.at[pl.ds((i - _NBUF) * T + p * P, P)],
                dsems.at[sl, p],
            ).wait()

    for p in range(_NSPLIT):
        rows = slice(p * P, (p + 1) * P)
        chunk = gate_val[rows, :, None] * slot[rows, None, :]
        bufs[sl, pl.ds(p * P, P)] = chunk
        dbufs[sl, pl.ds(p * P, P)] = (chunk != 0.0).astype(jnp.int8)
        pltpu.make_async_copy(
            bufs.at[sl, pl.ds(p * P, P)],
            combine_hbm.at[pl.ds(i * T + p * P, P)],
            sems.at[sl, p],
        ).start()
        pltpu.make_async_copy(
            dbufs.at[sl, pl.ds(p * P, P)],
            dispatch_hbm.at[pl.ds(i * T + p * P, P)],
            dsems.at[sl, p],
        ).start()

    @pl.when(i == n - 1)
    def _final():
        for k in range(_NBUF):
            step = n - _NBUF + k

            @pl.when(step >= 0)
            def _():
                s2 = lax.rem(jnp.int32(step), _NBUF)
                for p in range(_NSPLIT):
                    pltpu.make_async_copy(
                        bufs.at[s2, pl.ds(p * P, P)],
                        combine_hbm.at[pl.ds(step * T + p * P, P)],
                        sems.at[s2, p],
                    ).wait()
                for p in range(_NSPLIT):
                    pltpu.make_async_copy(
                        dbufs.at[s2, pl.ds(p * P, P)],
                        dispatch_hbm.at[pl.ds(step * T + p * P, P)],
                        dsems.at[s2, p],
                    ).wait()


@functools.partial(jax.jit, static_argnames=("block_t",))
def _top1_gate(x, W, block_t=512):
    S, D = x.shape
    E = W.shape[1]
    import numpy as np
    C = max(int(np.ceil(S / E * 1.0)), 4)
    n = S // block_t

    out_shapes = (
        jax.ShapeDtypeStruct((S, 1), jnp.int32),
        jax.ShapeDtypeStruct((1, E), jnp.int32),
        jax.ShapeDtypeStruct((1, 1), jnp.float32),
        jax.ShapeDtypeStruct((S, E, C), jnp.float32),
        jax.ShapeDtypeStruct((S, E, C), jnp.int8),
    )
    return pl.pallas_call(
        _gate_kernel,
        grid=(n,),
        in_specs=[
            pl.BlockSpec((block_t, D), lambda i: (i, 0)),
            pl.BlockSpec((E, D), lambda i: (0, 0)),
        ],
        out_specs=(
            pl.BlockSpec((block_t, 1), lambda i: (i, 0)),
            pl.BlockSpec((1, E), lambda i: (0, 0)),
            pl.BlockSpec((1, 1), lambda i: (0, 0)),
            pl.BlockSpec(memory_space=pl.ANY),
            pl.BlockSpec(memory_space=pl.ANY),
        ),
        out_shape=out_shapes,
        scratch_shapes=[
            pltpu.VMEM((_NBUF, block_t, E, C), jnp.float32),
            pltpu.VMEM((_NBUF, block_t, E, C), jnp.int8),
            pltpu.SemaphoreType.DMA((_NBUF, _NSPLIT)),
            pltpu.SemaphoreType.DMA((_NBUF, _NSPLIT)),
            pltpu.VMEM((1, E), jnp.int32),
            pltpu.VMEM((1, E), jnp.float32),
        ],
    )(x, W.T)


def kernel(input, W):
    import numpy as np
    S, D = input.shape
    E = W.shape[1]
    C = max(int(np.ceil(S / E * 1.0)), 4)

    idx2, counts, laux, combine, dispatch_i8 = _top1_gate(input, W)

    mask1 = (idx2 == jnp.arange(E, dtype=jnp.int32)[None, :]).astype(jnp.int32)

    return (laux[0, 0], combine, dispatch_i8.astype(jnp.bool_), mask1,
            counts[0], idx2[:, 0])


# speedup vs baseline: 1.0921x; 1.0611x over previous
import functools

import jax
import jax.numpy as jnp
from jax import lax
from jax.experimental import pallas as pl
from jax.experimental.pallas import tpu as pltpu

_NBUF = 3
_NSPLIT = 16


def _gate_kernel(x_ref, w_ref,
                 idx_ref,
                 counts_ref, laux_ref, combine_hbm, dispatch_hbm,
                 bufs, dbufs, sems, dsems, base_ref, gsum_ref):
    i = pl.program_id(0)
    n = pl.num_programs(0)
    T = x_ref.shape[0]
    E = w_ref.shape[0]
    C = bufs.shape[3]

    @pl.when(i == 0)
    def _init():
        base_ref[...] = jnp.zeros_like(base_ref)
        gsum_ref[...] = jnp.zeros_like(gsum_ref)

    x = x_ref[...]
    wt = w_ref[...]
    logits = lax.dot_general(x, wt, (((1,), (1,)), ((), ())),
                             preferred_element_type=jnp.float32)

    lmax = jnp.max(logits, axis=1, keepdims=True)
    ex = jnp.exp(logits - lmax)
    gates = ex / jnp.sum(ex, axis=1, keepdims=True)

    gmax = jnp.max(gates, axis=1, keepdims=True)
    eiota = lax.broadcasted_iota(jnp.int32, (T, E), 1)
    idx = jnp.min(jnp.where(gates == gmax, eiota, E), axis=1, keepdims=True)

    mask = (eiota == idx).astype(jnp.int32)
    mask_f = mask.astype(jnp.float32)
    row = lax.broadcasted_iota(jnp.int32, (T, T), 0)
    col = lax.broadcasted_iota(jnp.int32, (T, T), 1)
    tri = (col <= row).astype(jnp.float32)
    csum = jnp.dot(tri, mask_f, preferred_element_type=jnp.float32)
    base = base_ref[...].astype(jnp.float32)
    loc = jnp.sum((csum - 1.0 + base) * mask_f,
                  axis=1, keepdims=True).astype(jnp.int32)

    new_base = base_ref[...] + jnp.sum(mask, axis=0, keepdims=True)
    base_ref[...] = new_base
    gsum_ref[...] = gsum_ref[...] + jnp.sum(gates, axis=0, keepdims=True)

    idx_ref[...] = lax.transpose(idx, (1, 0))
    counts_ref[...] = new_base
    S = n * T
    laux_ref[...] = jnp.sum(
        (gsum_ref[...] / S) * (new_base.astype(jnp.float32) / S),
        keepdims=True,
    ) * E

    gate_val = jnp.where(eiota == idx, gmax, 0.0)
    ciota = lax.broadcasted_iota(jnp.int32, (T, C), 1)
    slot = (ciota == loc).astype(jnp.float32)
    sl = lax.rem(i, _NBUF)
    P = T // _NSPLIT

    @pl.when(i >= _NBUF)
    def _drain():
        for p in range(_NSPLIT):
            pltpu.make_async_copy(
                bufs.at[sl, pl.ds(p * P, P)],
                combine_hbm.at[pl.ds((i - _NBUF) * T + p * P, P)],
                sems.at[sl, p],
            ).wait()
        for p in range(_NSPLIT):
            pltpu.make_async_copy(
                dbufs.at[sl, pl.ds(p * P, P)],
                dispatch_hbm.at[pl.ds((i - _NBUF) * T + p * P, P)],
                dsems.at[sl, p],
            ).wait()

    for p in range(_NSPLIT):
        rows = slice(p * P, (p + 1) * P)
        chunk = gate_val[rows, :, None] * slot[rows, None, :]
        bufs[sl, pl.ds(p * P, P)] = chunk
        dbufs[sl, pl.ds(p * P, P)] = (chunk != 0.0).astype(jnp.int8)
        pltpu.make_async_copy(
            bufs.at[sl, pl.ds(p * P, P)],
            combine_hbm.at[pl.ds(i * T + p * P, P)],
            sems.at[sl, p],
        ).start()
        pltpu.make_async_copy(
            dbufs.at[sl, pl.ds(p * P, P)],
            dispatch_hbm.at[pl.ds(i * T + p * P, P)],
            dsems.at[sl, p],
        ).start()

    @pl.when(i == n - 1)
    def _final():
        for k in range(_NBUF):
            step = n - _NBUF + k

            @pl.when(step >= 0)
            def _():
                s2 = lax.rem(jnp.int32(step), _NBUF)
                for p in range(_NSPLIT):
                    pltpu.make_async_copy(
                        bufs.at[s2, pl.ds(p * P, P)],
                        combine_hbm.at[pl.ds(step * T + p * P, P)],
                        sems.at[s2, p],
                    ).wait()
                for p in range(_NSPLIT):
                    pltpu.make_async_copy(
                        dbufs.at[s2, pl.ds(p * P, P)],
                        dispatch_hbm.at[pl.ds(step * T + p * P, P)],
                        dsems.at[s2, p],
                    ).wait()


@functools.partial(jax.jit, static_argnames=("block_t",))
def _top1_gate(x, W, block_t=512):
    S, D = x.shape
    E = W.shape[1]
    import numpy as np
    C = max(int(np.ceil(S / E * 1.0)), 4)
    n = S // block_t

    out_shapes = (
        jax.ShapeDtypeStruct((1, S), jnp.int32),
        jax.ShapeDtypeStruct((1, E), jnp.int32),
        jax.ShapeDtypeStruct((1, 1), jnp.float32),
        jax.ShapeDtypeStruct((S, E, C), jnp.float32),
        jax.ShapeDtypeStruct((S, E, C), jnp.int8),
    )
    return pl.pallas_call(
        _gate_kernel,
        grid=(n,),
        in_specs=[
            pl.BlockSpec((block_t, D), lambda i: (i, 0)),
            pl.BlockSpec((E, D), lambda i: (0, 0)),
        ],
        out_specs=(
            pl.BlockSpec((1, block_t), lambda i: (0, i)),
            pl.BlockSpec((1, E), lambda i: (0, 0)),
            pl.BlockSpec((1, 1), lambda i: (0, 0)),
            pl.BlockSpec(memory_space=pl.ANY),
            pl.BlockSpec(memory_space=pl.ANY),
        ),
        out_shape=out_shapes,
        scratch_shapes=[
            pltpu.VMEM((_NBUF, block_t, E, C), jnp.float32),
            pltpu.VMEM((_NBUF, block_t, E, C), jnp.int8),
            pltpu.SemaphoreType.DMA((_NBUF, _NSPLIT)),
            pltpu.SemaphoreType.DMA((_NBUF, _NSPLIT)),
            pltpu.VMEM((1, E), jnp.int32),
            pltpu.VMEM((1, E), jnp.float32),
        ],
    )(x, W.T)


def kernel(input, W):
    import numpy as np
    S, D = input.shape
    E = W.shape[1]
    C = max(int(np.ceil(S / E * 1.0)), 4)

    idx2, counts, laux, combine, dispatch_i8 = _top1_gate(input, W)

    idx = idx2[0]
    mask1 = (idx[:, None] == jnp.arange(E, dtype=jnp.int32)[None, :]).astype(jnp.int32)

    return (laux[0, 0], combine, dispatch_i8.astype(jnp.bool_), mask1,
            counts[0], idx)
